# baseline (speedup 1.0000x reference)
import functools

import jax
import jax.numpy as jnp
from jax import lax
from jax.experimental import pallas as pl
from jax.experimental.pallas import tpu as pltpu
from jax.experimental.pallas import tpu_sc as plsc

N_JETS = 1024
EPS = 1e-05

NC = 2
NS = 16
LANES = 16


def _make_sc_edge(N, E, T):
    NW = NC * NS
    E_PER = E // NW
    B = 128
    NB = E_PER // B
    BTAIL = E_PER - NB * B
    ROWS_PER_TILE = N // NS
    ZR = 125
    NZ = ROWS_PER_TILE // ZR
    WB = 8 * ((N // NS + 7) // 8)
    WB_LAST = N - (NS - 1) * WB

    mesh = plsc.VectorSubcoreMesh(core_axis_name="c", subcore_axis_name="s")

    @functools.partial(
        pl.kernel,
        mesh=mesh,
        out_type=jax.ShapeDtypeStruct((NC, N, T), jnp.float32),
        compiler_params=pltpu.CompilerParams(
            needs_layout_passes=False, use_tc_tiling_on_sc=False),
        scratch_types=[
            [pltpu.VMEM((B,), jnp.int32) for _ in range(2)],
            [pltpu.VMEM((B,), jnp.int32) for _ in range(2)],
            [pltpu.VMEM((B,), jnp.int32) for _ in range(2)],
            [pltpu.VMEM((B,), jnp.float32) for _ in range(2)],
            [pltpu.VMEM((B, T), jnp.float32) for _ in range(2)],
            [pltpu.VMEM((B, T), jnp.float32) for _ in range(2)],
            pltpu.VMEM((BTAIL,), jnp.int32),
            pltpu.VMEM((BTAIL,), jnp.int32),
            pltpu.VMEM((ZR, T), jnp.float32),
            pltpu.VMEM_SHARED((N, T), jnp.float32),
            [pltpu.SemaphoreType.DMA for _ in range(2)],
            [pltpu.SemaphoreType.DMA for _ in range(2)],
            [pltpu.SemaphoreType.DMA for _ in range(2)],
        ],
    )
    def sc_edge(pw_hbm, head_hbm, tail_hbm, func_hbm, out_hbm,
                head_in, tail_in, head_idx, w_v, func_in, src_v,
                head2_v, tail2_v, z_v, pf_sh,
                sem_in, sem_w, sem_sc):
        c = lax.axis_index("c")
        s = lax.axis_index("s")
        wid = c * NS + s
        base_r = s * ROWS_PER_TILE

        def _zrow(i, carry):
            z_v[i, :] = jnp.zeros((LANES,), jnp.float32)
            return carry
        lax.fori_loop(0, ZR, _zrow, 0)

        def _zchunk(k, carry):
            pltpu.sync_copy(z_v, pf_sh.at[pl.ds(base_r + k * ZR, ZR), :])
            return carry
        lax.fori_loop(0, NZ, _zchunk, 0)

        plsc.subcore_barrier()

        ebase = wid * E_PER

        def _issue_in(i, b):
            off = ebase + i * B
            pltpu.async_copy(head_hbm.at[pl.ds(off, B)], head_in[b], sem_in[b])
            pltpu.async_copy(tail_hbm.at[pl.ds(off, B)], tail_in[b], sem_in[b])
            pltpu.async_copy(func_hbm.at[pl.ds(off, B), :], func_in[b],
                             sem_in[b])

        def _wait_in(i, b):
            off = ebase + i * B
            pltpu.make_async_copy(head_hbm.at[pl.ds(off, B)], head_in[b],
                                  sem_in[b]).wait()
            pltpu.make_async_copy(tail_hbm.at[pl.ds(off, B)], tail_in[b],
                                  sem_in[b]).wait()
            pltpu.make_async_copy(func_hbm.at[pl.ds(off, B), :], func_in[b],
                                  sem_in[b]).wait()

        def _wait_sc(b):
            pltpu.make_async_copy(src_v[b], pf_sh.at[head_idx[b]],
                                  sem_sc[b]).wait()

        def _do_batch(i, b):
            _wait_in(i, b)
            pltpu.async_copy(pw_hbm.at[tail_in[b]], w_v[b], sem_w[b])

            @pl.when(i >= 2)
            def _():
                _wait_sc(b)

            for g in range(B // LANES):
                sl = pl.ds(g * LANES, LANES)
                head_idx[b][sl] = head_in[b][sl]
            pltpu.make_async_copy(pw_hbm.at[tail_in[b]], w_v[b],
                                  sem_w[b]).wait()

            @plsc.parallel_loop(0, B, unroll=8)
            def _edges(e):
                wv = plsc.load_gather(
                    w_v[b], [jnp.full((LANES,), e, jnp.int32)])
                src_v[b][e, :] = func_in[b][e, :] * wv
            pltpu.async_copy(src_v[b], pf_sh.at[head_idx[b]], sem_sc[b],
                             add=True)

            @pl.when(i + 2 < NB)
            def _():
                _issue_in(i + 2, b)

        _issue_in(0, 0)
        if NB > 1:
            _issue_in(1, 1)

        def _pair(k, carry):
            _do_batch(2 * k, 0)
            _do_batch(2 * k + 1, 1)
            return carry
        lax.fori_loop(0, NB // 2, _pair, 0)
        if NB % 2:
            _do_batch(jnp.int32(NB - 1), (NB - 1) % 2)

        _wait_sc((NB - 1) % 2)
        if NB > 1:
            _wait_sc((NB - 2) % 2)

        if BTAIL:
            off = ebase + NB * B
            pltpu.sync_copy(head_hbm.at[pl.ds(off, BTAIL)], head2_v)
            pltpu.sync_copy(tail_hbm.at[pl.ds(off, BTAIL)], tail2_v)
            pltpu.sync_copy(func_hbm.at[pl.ds(off, BTAIL), :],
                            func_in[0].at[pl.ds(0, BTAIL), :])
            pltpu.sync_copy(pw_hbm.at[tail2_v], w_v[0].at[pl.ds(0, BTAIL)])

            def _edges2(e, carry):
                wv = plsc.load_gather(
                    w_v[0], [jnp.full((LANES,), e, jnp.int32)])
                src_v[0][e, :] = func_in[0][e, :] * wv
                return carry
            lax.fori_loop(0, BTAIL, _edges2, 0)
            pltpu.sync_copy(src_v[0].at[pl.ds(0, BTAIL), :],
                            pf_sh.at[head2_v], add=True)

        plsc.subcore_barrier()

        @pl.when(s < NS - 1)
        def _():
            pltpu.sync_copy(pf_sh.at[pl.ds(s * WB, WB), :],
                            out_hbm.at[c, pl.ds(s * WB, WB), :])

        @pl.when(s == NS - 1)
        def _():
            pltpu.sync_copy(pf_sh.at[pl.ds((NS - 1) * WB, WB_LAST), :],
                            out_hbm.at[c, pl.ds((NS - 1) * WB, WB_LAST), :])

    return sc_edge


def _stats_body(nblk, pf_ref, w_ref, ind_ref, mean_ref, rstd_ref, jet_acc):
    j = pl.program_id(0)

    @pl.when(j == 0)
    def _():
        jet_acc[...] = jnp.zeros_like(jet_acc)

    pf = pf_ref[0] + pf_ref[1]
    wgt = pf * w_ref[...]
    ind = ind_ref[0]
    R = ind.shape[1]
    oh = (lax.broadcasted_iota(jnp.int32, (N_JETS, R), 0) == ind
          ).astype(jnp.float32)
    jet_acc[...] += jnp.dot(oh, wgt, preferred_element_type=jnp.float32)

    @pl.when(j == nblk - 1)
    def _():
        jet = jet_acc[...]
        bs = ind_ref[0, 0, R - 1] + 1
        b_f = bs.astype(jnp.float32)
        mean = jnp.sum(jet, axis=0, keepdims=True) / b_f
        mask = (lax.broadcasted_iota(jnp.int32, (N_JETS, 1), 0) < bs
                ).astype(jnp.float32)
        var = jnp.sum(mask * jnp.square(jet - mean), axis=0,
                      keepdims=True) / (b_f - 1.0)
        mean_ref[...] = mean
        rstd_ref[...] = lax.rsqrt(var + EPS)


def _main_body(pf_ref, mean_ref, rstd_ref, W_ref, b_ref, out_ref):
    pf = pf_ref[0] + pf_ref[1]
    normed = (pf - mean_ref[...]) * rstd_ref[...]
    out_ref[...] = lax.dot_general(
        normed, W_ref[...], (((1,), (1,)), ((), ())),
        preferred_element_type=jnp.float32) + b_ref[...]


def kernel(part_weight, pair_head, pair_tail, pair_func, part_indicator, W, b):
    N = part_weight.shape[0]
    E = pair_head.shape[0]
    T = pair_func.shape[1]
    C = W.shape[0]

    pw = part_weight[:, 0]
    head = pair_head[:, 0]
    tail = pair_tail[:, 0]

    pf_partial = _make_sc_edge(N, E, T)(pw, head, tail, pair_func)

    R = 800
    nblk = N // R

    mean, rstd = pl.pallas_call(
        functools.partial(_stats_body, nblk),
        grid=(nblk,),
        in_specs=[
            pl.BlockSpec((NC, R, T), lambda j: (0, j, 0)),
            pl.BlockSpec((R, 1), lambda j: (j, 0)),
            pl.BlockSpec((1, 1, R), lambda j: (j, 0, 0)),
        ],
        out_specs=[
            pl.BlockSpec((1, T), lambda j: (0, 0)),
            pl.BlockSpec((1, T), lambda j: (0, 0)),
        ],
        out_shape=[
            jax.ShapeDtypeStruct((1, T), jnp.float32),
            jax.ShapeDtypeStruct((1, T), jnp.float32),
        ],
        scratch_shapes=[pltpu.VMEM((N_JETS, T), jnp.float32)],
    )(pf_partial, part_weight, part_indicator.reshape(nblk, 1, R))

    out = pl.pallas_call(
        _main_body,
        grid=(nblk,),
        in_specs=[
            pl.BlockSpec((NC, R, T), lambda j: (0, j, 0)),
            pl.BlockSpec((1, T), lambda j: (0, 0)),
            pl.BlockSpec((1, T), lambda j: (0, 0)),
            pl.BlockSpec((C, T), lambda j: (0, 0)),
            pl.BlockSpec((1, C), lambda j: (0, 0)),
        ],
        out_specs=pl.BlockSpec((R, C), lambda j: (j, 0)),
        out_shape=jax.ShapeDtypeStruct((N, C), jnp.float32),
    )(pf_partial, mean, rstd, W, b.reshape(1, C))

    return out


# speedup vs baseline: 1.2671x; 1.2671x over previous
import functools

import jax
import jax.numpy as jnp
from jax import lax
from jax.experimental import pallas as pl
from jax.experimental.pallas import tpu as pltpu
from jax.experimental.pallas import tpu_sc as plsc

N_JETS = 1024
EPS = 1e-05

NC = 2
NS = 16
LANES = 16


def _make_sc_edge(N, E, T):
    RW = 128
    KR = 2
    CB = KR * RW
    TOT_ROWS = E // RW
    ROWS_PER_CORE = TOT_ROWS // NC
    ROWS0 = 4 * ((ROWS_PER_CORE // NS + 3) // 4)
    ROWS_LAST = ROWS_PER_CORE - (NS - 1) * ROWS0
    NCH0 = ROWS0 // KR
    NCH_LAST = ROWS_LAST // KR
    assert ROWS_LAST > 0 and ROWS_LAST % (2 * KR) == 0 and ROWS0 % (2 * KR) == 0
    ROWS_PER_TILE = N // NS
    ZR = 125
    NZ = ROWS_PER_TILE // ZR
    WB = 8 * ((N // NS + 7) // 8)
    WB_LAST = N - (NS - 1) * WB

    mesh = plsc.VectorSubcoreMesh(core_axis_name="c", subcore_axis_name="s")

    @functools.partial(
        pl.kernel,
        mesh=mesh,
        out_type=jax.ShapeDtypeStruct((NC, N, T), jnp.float32),
        compiler_params=pltpu.CompilerParams(
            needs_layout_passes=False, use_tc_tiling_on_sc=False),
        scratch_types=[
            [pltpu.VMEM((KR, RW), jnp.int32) for _ in range(2)],
            [pltpu.VMEM((KR, RW), jnp.int32) for _ in range(2)],
            [pltpu.VMEM((KR, RW), jnp.int32) for _ in range(2)],
            [pltpu.VMEM((KR, RW), jnp.float32) for _ in range(2)],
            [pltpu.VMEM((CB, T), jnp.float32) for _ in range(2)],
            [pltpu.VMEM((KR, RW, T), jnp.float32) for _ in range(2)],
            pltpu.VMEM((ZR, T), jnp.float32),
            pltpu.VMEM_SHARED((N,), jnp.float32),
            pltpu.VMEM_SHARED((N, T), jnp.float32),
            [pltpu.SemaphoreType.DMA for _ in range(2)],
            [pltpu.SemaphoreType.DMA for _ in range(2)],
            [pltpu.SemaphoreType.DMA for _ in range(2)],
        ],
    )
    def sc_edge(pw_hbm, head_hbm, tail_hbm, func_hbm, out_hbm,
                head_in, tail_in, head_idx, w_v, func_in, src_v,
                z_v, pw_sh, pf_sh,
                sem_in, sem_w, sem_sc):
        c = lax.axis_index("c")
        s = lax.axis_index("s")

        base_r = s * ROWS_PER_TILE

        @pl.when(s < NS - 1)
        def _():
            pltpu.sync_copy(pw_hbm.at[pl.ds(s * WB, WB)],
                            pw_sh.at[pl.ds(s * WB, WB)])

        @pl.when(s == NS - 1)
        def _():
            pltpu.sync_copy(pw_hbm.at[pl.ds((NS - 1) * WB, WB_LAST)],
                            pw_sh.at[pl.ds((NS - 1) * WB, WB_LAST)])

        def _zrow(i, carry):
            z_v[i, :] = jnp.zeros((LANES,), jnp.float32)
            return carry
        lax.fori_loop(0, ZR, _zrow, 0)

        def _zchunk(k, carry):
            pltpu.sync_copy(z_v, pf_sh.at[pl.ds(base_r + k * ZR, ZR), :])
            return carry
        lax.fori_loop(0, NZ, _zchunk, 0)

        plsc.subcore_barrier()

        row_base = c * ROWS_PER_CORE + s * ROWS0
        nch = jnp.where(s < NS - 1, NCH0, NCH_LAST)

        def _issue_in(i, b):
            roff = row_base + i * KR
            pltpu.async_copy(head_hbm.at[pl.ds(roff, KR), :], head_in[b],
                             sem_in[b])
            pltpu.async_copy(tail_hbm.at[pl.ds(roff, KR), :], tail_in[b],
                             sem_in[b])
            pltpu.async_copy(func_hbm.at[pl.ds(roff * RW, CB), :], func_in[b],
                             sem_in[b])

        def _wait_in(i, b):
            roff = row_base + i * KR
            pltpu.make_async_copy(head_hbm.at[pl.ds(roff, KR), :], head_in[b],
                                  sem_in[b]).wait()
            pltpu.make_async_copy(tail_hbm.at[pl.ds(roff, KR), :], tail_in[b],
                                  sem_in[b]).wait()
            pltpu.make_async_copy(func_hbm.at[pl.ds(roff * RW, CB), :],
                                  func_in[b], sem_in[b]).wait()

        def _wait_sc(b):
            for k in range(KR):
                pltpu.make_async_copy(src_v[b].at[k],
                                      pf_sh.at[head_idx[b].at[k]],
                                      sem_sc[b]).wait()

        def _do_batch(i, b):
            _wait_in(i, b)
            for k in range(KR):
                pltpu.async_copy(pw_sh.at[tail_in[b].at[k]], w_v[b].at[k],
                                 sem_w[b])

            @pl.when(i >= 2)
            def _():
                _wait_sc(b)

            for k in range(KR):
                for g in range(RW // LANES):
                    sl = pl.ds(g * LANES, LANES)
                    head_idx[b][k, sl] = head_in[b][k, sl]
            for k in range(KR):
                pltpu.make_async_copy(pw_sh.at[tail_in[b].at[k]],
                                      w_v[b].at[k], sem_w[b]).wait()

            for k in range(KR):
                kvec = jnp.full((LANES,), k, jnp.int32)

                @plsc.parallel_loop(0, RW, unroll=8)
                def _edges(e):
                    wv = plsc.load_gather(
                        w_v[b], [kvec, jnp.full((LANES,), e, jnp.int32)])
                    src_v[b][k, e, :] = func_in[b][k * RW + e, :] * wv
            for k in range(KR):
                pltpu.async_copy(src_v[b].at[k], pf_sh.at[head_idx[b].at[k]],
                                 sem_sc[b], add=True)

            @pl.when(i + 2 < nch)
            def _():
                _issue_in(i + 2, b)

        _issue_in(0, 0)
        _issue_in(1, 1)

        def _pair(k, carry):
            _do_batch(2 * k, 0)
            _do_batch(2 * k + 1, 1)
            return carry
        lax.fori_loop(0, nch // 2, _pair, 0)

        _wait_sc(0)
        _wait_sc(1)

        plsc.subcore_barrier()

        @pl.when(s < NS - 1)
        def _():
            pltpu.sync_copy(pf_sh.at[pl.ds(s * WB, WB), :],
                            out_hbm.at[c, pl.ds(s * WB, WB), :])

        @pl.when(s == NS - 1)
        def _():
            pltpu.sync_copy(pf_sh.at[pl.ds((NS - 1) * WB, WB_LAST), :],
                            out_hbm.at[c, pl.ds((NS - 1) * WB, WB_LAST), :])

    return sc_edge


def _stats_body(nblk, pf_ref, w_ref, ind_ref, mean_ref, rstd_ref, jet_acc):
    j = pl.program_id(0)

    @pl.when(j == 0)
    def _():
        jet_acc[...] = jnp.zeros_like(jet_acc)

    pf = pf_ref[0] + pf_ref[1]
    wgt = pf * w_ref[...]
    ind = ind_ref[0]
    R = ind.shape[1]
    oh = (lax.broadcasted_iota(jnp.int32, (N_JETS, R), 0) == ind
          ).astype(jnp.float32)
    jet_acc[...] += jnp.dot(oh, wgt, preferred_element_type=jnp.float32)

    @pl.when(j == nblk - 1)
    def _():
        jet = jet_acc[...]
        bs = ind_ref[0, 0, R - 1] + 1
        b_f = bs.astype(jnp.float32)
        mean = jnp.sum(jet, axis=0, keepdims=True) / b_f
        mask = (lax.broadcasted_iota(jnp.int32, (N_JETS, 1), 0) < bs
                ).astype(jnp.float32)
        var = jnp.sum(mask * jnp.square(jet - mean), axis=0,
                      keepdims=True) / (b_f - 1.0)
        mean_ref[...] = mean
        rstd_ref[...] = lax.rsqrt(var + EPS)


def _main_body(pf_ref, mean_ref, rstd_ref, W_ref, b_ref, out_ref):
    pf = pf_ref[0] + pf_ref[1]
    normed = (pf - mean_ref[...]) * rstd_ref[...]
    out_ref[...] = lax.dot_general(
        normed, W_ref[...], (((1,), (1,)), ((), ())),
        preferred_element_type=jnp.float32) + b_ref[...]


def kernel(part_weight, pair_head, pair_tail, pair_func, part_indicator, W, b):
    N = part_weight.shape[0]
    E = pair_head.shape[0]
    T = pair_func.shape[1]
    C = W.shape[0]

    pw = part_weight[:, 0]
    head = pair_head[:, 0].reshape(E // 128, 128)
    tail = pair_tail[:, 0].reshape(E // 128, 128)

    pf_partial = _make_sc_edge(N, E, T)(pw, head, tail, pair_func)

    R = 800
    nblk = N // R

    mean, rstd = pl.pallas_call(
        functools.partial(_stats_body, nblk),
        grid=(nblk,),
        in_specs=[
            pl.BlockSpec((NC, R, T), lambda j: (0, j, 0)),
            pl.BlockSpec((R, 1), lambda j: (j, 0)),
            pl.BlockSpec((1, 1, R), lambda j: (j, 0, 0)),
        ],
        out_specs=[
            pl.BlockSpec((1, T), lambda j: (0, 0)),
            pl.BlockSpec((1, T), lambda j: (0, 0)),
        ],
        out_shape=[
            jax.ShapeDtypeStruct((1, T), jnp.float32),
            jax.ShapeDtypeStruct((1, T), jnp.float32),
        ],
        scratch_shapes=[pltpu.VMEM((N_JETS, T), jnp.float32)],
    )(pf_partial, part_weight, part_indicator.reshape(nblk, 1, R))

    out = pl.pallas_call(
        _main_body,
        grid=(nblk,),
        in_specs=[
            pl.BlockSpec((NC, R, T), lambda j: (0, j, 0)),
            pl.BlockSpec((1, T), lambda j: (0, 0)),
            pl.BlockSpec((1, T), lambda j: (0, 0)),
            pl.BlockSpec((C, T), lambda j: (0, 0)),
            pl.BlockSpec((1, C), lambda j: (0, 0)),
        ],
        out_specs=pl.BlockSpec((R, C), lambda j: (j, 0)),
        out_shape=jax.ShapeDtypeStruct((N, C), jnp.float32),
    )(pf_partial, mean, rstd, W, b.reshape(1, C))

    return out
